# Initial kernel scaffold; baseline (speedup 1.0000x reference)
import functools

import jax
import jax.numpy as jnp
from jax import lax
from jax.experimental import pallas as pl
from jax.experimental.pallas import tpu as pltpu
from jax.experimental.pallas import tpu_sc as plsc

NUM_FIELDS = 9
EMB_DIM = 16
NUM_CORES = 2
NUM_SUBCORES = 16
NUM_WORKERS = NUM_CORES * NUM_SUBCORES
CHUNK = 128


def _sc_body(rows_per_worker, *refs):
    nchunk = rows_per_worker // CHUNK
    idx_hbm = refs[0:9]
    emb_hbm = refs[9:18]
    lin_hbm = refs[18:27]
    eout = refs[27:36]
    lsum_out = refs[36]
    idx_v = refs[37]
    ebufs = refs[38:47]
    lbufs = refs[47:56]
    lacc = refs[56]
    sem = refs[57]

    c = lax.axis_index("c")
    s = lax.axis_index("s")
    wid = s * NUM_CORES + c
    rbase = wid * rows_per_worker
    cbase = wid * nchunk

    for f in range(NUM_FIELDS):
        pltpu.sync_copy(idx_hbm[f].at[pl.ds(cbase, nchunk)], idx_v.at[f])

    descs = []
    for f in range(NUM_FIELDS):
        for ch in range(nchunk):
            descs.append(pltpu.async_copy(
                emb_hbm[f].at[idx_v.at[f, ch]],
                ebufs[f].at[pl.ds(ch * CHUNK, CHUNK)], sem))
            descs.append(pltpu.async_copy(
                lin_hbm[f].at[idx_v.at[f, ch]],
                lbufs[f].at[pl.ds(ch * CHUNK, CHUNK)], sem))
    for dsc in descs:
        dsc.wait()

    for i in range(rows_per_worker // 16):
        sl = pl.ds(i * 16, 16)
        acc = lbufs[0][sl]
        for f in range(1, NUM_FIELDS):
            acc = acc + lbufs[f][sl]
        lacc[sl] = acc

    for f in range(NUM_FIELDS):
        pltpu.sync_copy(ebufs[f], eout[f].at[pl.ds(rbase, rows_per_worker)])
    pltpu.sync_copy(lacc, lsum_out.at[pl.ds(rbase, rows_per_worker)])


def _sc_gather(idx2d, embs, lin1d):
    batch = idx2d[0].shape[0] * CHUNK
    rows_per_worker = batch // NUM_WORKERS
    out_type = ([jax.ShapeDtypeStruct((batch, EMB_DIM), jnp.float32)
                 for _ in range(NUM_FIELDS)]
                + [jax.ShapeDtypeStruct((batch,), jnp.float32)])
    scratch = ([pltpu.VMEM((NUM_FIELDS, rows_per_worker // CHUNK, CHUNK),
                           jnp.int32)]
               + [pltpu.VMEM((rows_per_worker, EMB_DIM), jnp.float32)
                  for _ in range(NUM_FIELDS)]
               + [pltpu.VMEM((rows_per_worker,), jnp.float32)
                  for _ in range(NUM_FIELDS)]
               + [pltpu.VMEM((rows_per_worker,), jnp.float32),
                  pltpu.SemaphoreType.DMA])
    fn = pl.kernel(
        functools.partial(_sc_body, rows_per_worker),
        out_type=out_type,
        mesh=plsc.VectorSubcoreMesh(core_axis_name="c", subcore_axis_name="s"),
        scratch_types=scratch,
    )
    return fn(*idx2d, *embs, *lin1d)


def _dot(a, b):
    return jax.lax.dot_general(
        a, b, (((1,), (0,)), ((), ())),
        precision=jax.lax.Precision.HIGHEST,
        preferred_element_type=jnp.float32)


def _tc_body(*refs):
    e_refs = refs[0:9]
    (xn_ref, ls_ref, w0e_ref, w0n_ref, b0_ref, w1_ref, b1_ref,
     lnw_ref, lnb_ref, wfm_ref, wlin_ref, owh_ref, ob_ref, out_ref) = refs[9:]

    es = [r[...] for r in e_refs]
    sum_e = es[0]
    for e in es[1:]:
        sum_e = sum_e + e
    sq_sum = jnp.sum(es[0] * es[0], axis=1, keepdims=True)
    for e in es[1:]:
        sq_sum = sq_sum + jnp.sum(e * e, axis=1, keepdims=True)
    fm = 0.5 * (jnp.sum(sum_e * sum_e, axis=1, keepdims=True) - sq_sum)

    xn = xn_ref[...]
    lin = ls_ref[...] + jnp.sum(xn * lnw_ref[...], axis=1, keepdims=True) \
        + lnb_ref[...]

    w0e = w0e_ref[...]
    h = _dot(xn, w0n_ref[...]) + b0_ref[...]
    for f in range(NUM_FIELDS):
        h = h + _dot(es[f], w0e[f * EMB_DIM:(f + 1) * EMB_DIM, :])
    h = jnp.maximum(h, 0.0)
    h = jnp.maximum(_dot(h, w1_ref[...]) + b1_ref[...], 0.0)

    logit = (fm * wfm_ref[...] + lin * wlin_ref[...]
             + jnp.sum(h * owh_ref[...], axis=1, keepdims=True) + ob_ref[...])
    out_ref[...] = jax.nn.sigmoid(logit)


def _tc_specs(batch, blk):
    def rowblock(shape1):
        return pl.BlockSpec((blk, shape1), lambda i: (i, 0))

    def whole(shape):
        return pl.BlockSpec(shape, lambda i: (0, 0))

    in_specs = ([rowblock(EMB_DIM) for _ in range(NUM_FIELDS)]
                + [rowblock(3), rowblock(1),
                   whole((NUM_FIELDS * EMB_DIM, 64)), whole((3, 64)),
                   whole((1, 64)), whole((64, 32)), whole((1, 32)),
                   whole((1, 3)), whole((1, 1)), whole((1, 1)),
                   whole((1, 1)), whole((1, 32)), whole((1, 1))])
    out_specs = rowblock(1)
    return (batch // blk,), in_specs, out_specs


def _tc_dense(es, xn, ls2d, w0e, w0n, b0, w1, b1, lnw, lnb, wfm, wlin, owh,
              ob, blk=1024):
    batch = xn.shape[0]
    grid, in_specs, out_specs = _tc_specs(batch, blk)
    return pl.pallas_call(
        _tc_body,
        grid=grid,
        in_specs=in_specs,
        out_specs=out_specs,
        out_shape=jax.ShapeDtypeStruct((batch, 1), jnp.float32),
    )(*es, xn, ls2d, w0e, w0n, b0, w1, b1, lnw, lnb, wfm, wlin, owh, ob)


def kernel(idx_user_id, idx_region, idx_device, idx_gender, idx_banner_id,
           idx_brand, idx_vertical, idx_language, idx_price_tier,
           x_num,
           emb_user_id, emb_region, emb_device, emb_gender, emb_banner_id,
           emb_brand, emb_vertical, emb_language, emb_price_tier,
           lin_user_id, lin_region, lin_device, lin_gender, lin_banner_id,
           lin_brand, lin_vertical, lin_language, lin_price_tier,
           lin_num_W, lin_num_b,
           dnn_W0, dnn_b0, dnn_W1, dnn_b1,
           out_W, out_b):
    idxs = [idx_user_id, idx_region, idx_device, idx_gender, idx_banner_id,
            idx_brand, idx_vertical, idx_language, idx_price_tier]
    embs = [emb_user_id, emb_region, emb_device, emb_gender, emb_banner_id,
            emb_brand, emb_vertical, emb_language, emb_price_tier]
    lins = [lin_user_id, lin_region, lin_device, lin_gender, lin_banner_id,
            lin_brand, lin_vertical, lin_language, lin_price_tier]
    batch = idxs[0].shape[0]

    idx2d = [jnp.reshape(i.astype(jnp.int32), (batch // CHUNK, CHUNK))
             for i in idxs]
    lin1d = [jnp.reshape(l, (-1,)) for l in lins]

    sc_out = _sc_gather(idx2d, embs, lin1d)
    es, lsum = list(sc_out[:NUM_FIELDS]), sc_out[NUM_FIELDS]

    nd = NUM_FIELDS * EMB_DIM
    return _tc_dense(
        es, x_num, jnp.reshape(lsum, (batch, 1)),
        dnn_W0[:, :nd].T, dnn_W0[:, nd:].T, jnp.reshape(dnn_b0, (1, -1)),
        dnn_W1.T, jnp.reshape(dnn_b1, (1, -1)),
        lin_num_W, jnp.reshape(lin_num_b, (1, 1)),
        out_W[:, 0:1], out_W[:, 1:2], out_W[:, 2:],
        jnp.reshape(out_b, (1, 1)))


# speedup vs baseline: 2.6567x; 2.6567x over previous
import functools

import jax
import jax.numpy as jnp
from jax import lax
from jax.experimental import pallas as pl
from jax.experimental.pallas import tpu as pltpu
from jax.experimental.pallas import tpu_sc as plsc

NUM_FIELDS = 9
EMB_DIM = 16
NUM_CORES = 2
NUM_SUBCORES = 16
NUM_WORKERS = NUM_CORES * NUM_SUBCORES
CHUNK = 128


def _sc_body(rows_per_worker, *refs):
    nchunk = rows_per_worker // CHUNK
    idx_hbm = refs[0:9]
    emb_hbm = refs[9:18]
    lin_hbm = refs[18:27]
    eout = refs[27:36]
    lsum_out = refs[36]
    idx_v = refs[37]
    ebufs = refs[38:47]
    lbufs = refs[47:56]
    lacc = refs[56]
    sem = refs[57]

    c = lax.axis_index("c")
    s = lax.axis_index("s")
    wid = s * NUM_CORES + c
    rbase = wid * rows_per_worker
    cbase = wid * nchunk

    for f in range(NUM_FIELDS):
        pltpu.sync_copy(idx_hbm[f].at[pl.ds(cbase, nchunk)], idx_v.at[f])

    descs = []
    for f in range(NUM_FIELDS):
        for ch in range(nchunk):
            descs.append(pltpu.async_copy(
                emb_hbm[f].at[idx_v.at[f, ch]],
                ebufs[f].at[pl.ds(ch * CHUNK, CHUNK)], sem))
            descs.append(pltpu.async_copy(
                lin_hbm[f].at[idx_v.at[f, ch]],
                lbufs[f].at[pl.ds(ch * CHUNK, CHUNK)], sem))
    for dsc in descs:
        dsc.wait()

    for i in range(rows_per_worker // 16):
        sl = pl.ds(i * 16, 16)
        acc = lbufs[0][sl]
        for f in range(1, NUM_FIELDS):
            acc = acc + lbufs[f][sl]
        lacc[sl] = acc

    for f in range(NUM_FIELDS):
        pltpu.sync_copy(ebufs[f], eout[f].at[pl.ds(rbase, rows_per_worker)])
    pltpu.sync_copy(lacc, lsum_out.at[pl.ds(rbase, rows_per_worker)])


def _sc_gather(idx2d, embs, lin1d):
    batch = idx2d[0].shape[0] * CHUNK
    rows_per_worker = batch // NUM_WORKERS
    out_type = ([jax.ShapeDtypeStruct((batch, EMB_DIM), jnp.float32)
                 for _ in range(NUM_FIELDS)]
                + [jax.ShapeDtypeStruct((batch,), jnp.float32)])
    scratch = ([pltpu.VMEM((NUM_FIELDS, rows_per_worker // CHUNK, CHUNK),
                           jnp.int32)]
               + [pltpu.VMEM((rows_per_worker, EMB_DIM), jnp.float32)
                  for _ in range(NUM_FIELDS)]
               + [pltpu.VMEM((rows_per_worker,), jnp.float32)
                  for _ in range(NUM_FIELDS)]
               + [pltpu.VMEM((rows_per_worker,), jnp.float32),
                  pltpu.SemaphoreType.DMA])
    fn = pl.kernel(
        functools.partial(_sc_body, rows_per_worker),
        out_type=out_type,
        mesh=plsc.VectorSubcoreMesh(core_axis_name="c", subcore_axis_name="s"),
        scratch_types=scratch,
        compiler_params=pltpu.CompilerParams(use_tc_tiling_on_sc=False),
    )
    return fn(*idx2d, *embs, *lin1d)


def _dot(a, b):
    return jax.lax.dot_general(
        a, b, (((1,), (0,)), ((), ())),
        precision=jax.lax.Precision.HIGHEST,
        preferred_element_type=jnp.float32)


def _tc_body(*refs):
    e_refs = refs[0:9]
    (xn_ref, ls_ref, w0e_ref, w0n_ref, b0_ref, w1_ref, b1_ref,
     lnw_ref, lnb_ref, wfm_ref, wlin_ref, owh_ref, ob_ref, out_ref) = refs[9:]

    es = [r[...] for r in e_refs]
    sum_e = es[0]
    for e in es[1:]:
        sum_e = sum_e + e
    sq_sum = jnp.sum(es[0] * es[0], axis=1, keepdims=True)
    for e in es[1:]:
        sq_sum = sq_sum + jnp.sum(e * e, axis=1, keepdims=True)
    fm = 0.5 * (jnp.sum(sum_e * sum_e, axis=1, keepdims=True) - sq_sum)

    xn = xn_ref[...]
    lin = ls_ref[...] + jnp.sum(xn * lnw_ref[...], axis=1, keepdims=True) \
        + lnb_ref[...]

    w0e = w0e_ref[...]
    h = _dot(xn, w0n_ref[...]) + b0_ref[...]
    for f in range(NUM_FIELDS):
        h = h + _dot(es[f], w0e[f * EMB_DIM:(f + 1) * EMB_DIM, :])
    h = jnp.maximum(h, 0.0)
    h = jnp.maximum(_dot(h, w1_ref[...]) + b1_ref[...], 0.0)

    logit = (fm * wfm_ref[...] + lin * wlin_ref[...]
             + jnp.sum(h * owh_ref[...], axis=1, keepdims=True) + ob_ref[...])
    out_ref[...] = jax.nn.sigmoid(logit)


def _tc_specs(batch, blk):
    def rowblock(shape1):
        return pl.BlockSpec((blk, shape1), lambda i: (i, 0))

    def whole(shape):
        return pl.BlockSpec(shape, lambda i: (0, 0))

    in_specs = ([rowblock(EMB_DIM) for _ in range(NUM_FIELDS)]
                + [rowblock(3), rowblock(1),
                   whole((NUM_FIELDS * EMB_DIM, 64)), whole((3, 64)),
                   whole((1, 64)), whole((64, 32)), whole((1, 32)),
                   whole((1, 3)), whole((1, 1)), whole((1, 1)),
                   whole((1, 1)), whole((1, 32)), whole((1, 1))])
    out_specs = rowblock(1)
    return (batch // blk,), in_specs, out_specs


def _tc_dense(es, xn, ls2d, w0e, w0n, b0, w1, b1, lnw, lnb, wfm, wlin, owh,
              ob, blk=1024):
    batch = xn.shape[0]
    grid, in_specs, out_specs = _tc_specs(batch, blk)
    return pl.pallas_call(
        _tc_body,
        grid=grid,
        in_specs=in_specs,
        out_specs=out_specs,
        out_shape=jax.ShapeDtypeStruct((batch, 1), jnp.float32),
    )(*es, xn, ls2d, w0e, w0n, b0, w1, b1, lnw, lnb, wfm, wlin, owh, ob)


def kernel(idx_user_id, idx_region, idx_device, idx_gender, idx_banner_id,
           idx_brand, idx_vertical, idx_language, idx_price_tier,
           x_num,
           emb_user_id, emb_region, emb_device, emb_gender, emb_banner_id,
           emb_brand, emb_vertical, emb_language, emb_price_tier,
           lin_user_id, lin_region, lin_device, lin_gender, lin_banner_id,
           lin_brand, lin_vertical, lin_language, lin_price_tier,
           lin_num_W, lin_num_b,
           dnn_W0, dnn_b0, dnn_W1, dnn_b1,
           out_W, out_b):
    idxs = [idx_user_id, idx_region, idx_device, idx_gender, idx_banner_id,
            idx_brand, idx_vertical, idx_language, idx_price_tier]
    embs = [emb_user_id, emb_region, emb_device, emb_gender, emb_banner_id,
            emb_brand, emb_vertical, emb_language, emb_price_tier]
    lins = [lin_user_id, lin_region, lin_device, lin_gender, lin_banner_id,
            lin_brand, lin_vertical, lin_language, lin_price_tier]
    batch = idxs[0].shape[0]

    idx2d = [jnp.reshape(i.astype(jnp.int32), (batch // CHUNK, CHUNK))
             for i in idxs]
    lin1d = [jnp.reshape(l, (-1,)) for l in lins]

    sc_out = _sc_gather(idx2d, embs, lin1d)
    es, lsum = list(sc_out[:NUM_FIELDS]), sc_out[NUM_FIELDS]

    nd = NUM_FIELDS * EMB_DIM
    return _tc_dense(
        es, x_num, jnp.reshape(lsum, (batch, 1)),
        dnn_W0[:, :nd].T, dnn_W0[:, nd:].T, jnp.reshape(dnn_b0, (1, -1)),
        dnn_W1.T, jnp.reshape(dnn_b1, (1, -1)),
        lin_num_W, jnp.reshape(lin_num_b, (1, 1)),
        out_W[:, 0:1], out_W[:, 1:2], out_W[:, 2:],
        jnp.reshape(out_b, (1, 1)))


# speedup vs baseline: 2.9689x; 1.1175x over previous
import functools

import jax
import jax.numpy as jnp
from jax import lax
from jax.experimental import pallas as pl
from jax.experimental.pallas import tpu as pltpu
from jax.experimental.pallas import tpu_sc as plsc

NUM_FIELDS = 9
EMB_DIM = 16
NUM_CORES = 2
NUM_SUBCORES = 16
NUM_WORKERS = NUM_CORES * NUM_SUBCORES
CHUNK = 128


def _sc_body(rows_per_worker, *refs):
    nchunk = rows_per_worker // CHUNK
    idx_hbm = refs[0:9]
    emb_hbm = refs[9:18]
    lin_hbm = refs[18:27]
    emain_out = refs[27]
    e8_out = refs[28]
    lsum_out = refs[29]
    idx_v = refs[30]
    estage = refs[31:40]
    lbufs = refs[40:49]
    lacc = refs[49]
    sem = refs[50]
    wsem = refs[51]

    c = lax.axis_index("c")
    s = lax.axis_index("s")
    wid = s * NUM_CORES + c
    rbase = wid * rows_per_worker
    rows = pl.ds(rbase, rows_per_worker)

    for f in range(NUM_FIELDS):
        pltpu.sync_copy(idx_hbm[f].at[rows], idx_v.at[f])

    egath, lgath = [], []
    for f in range(NUM_FIELDS):
        for ch in range(nchunk):
            cs = pl.ds(ch * CHUNK, CHUNK)
            egath.append(pltpu.async_copy(
                emb_hbm[f].at[idx_v.at[f, cs]], estage[f].at[cs], sem))
            lgath.append(pltpu.async_copy(
                lin_hbm[f].at[idx_v.at[f, cs]], lbufs[f].at[cs], sem))

    for dsc in egath:
        dsc.wait()
    writes = []
    for f in range(NUM_FIELDS - 1):
        writes.append(pltpu.async_copy(
            estage[f], emain_out.at[rows, pl.ds(f * EMB_DIM, EMB_DIM)],
            wsem))
    writes.append(pltpu.async_copy(estage[8], e8_out.at[rows], wsem))

    for dsc in lgath:
        dsc.wait()
    for i in range(rows_per_worker // 16):
        sl = pl.ds(i * 16, 16)
        acc = lbufs[0][sl]
        for f in range(1, NUM_FIELDS):
            acc = acc + lbufs[f][sl]
        lacc[sl] = acc
    pltpu.sync_copy(lacc, lsum_out.at[rows])
    for dsc in writes:
        dsc.wait()


def _sc_gather(idxs, embs, lins):
    batch = idxs[0].shape[0]
    rows_per_worker = batch // NUM_WORKERS
    out_type = [jax.ShapeDtypeStruct((batch, 128), jnp.float32),
                jax.ShapeDtypeStruct((batch, EMB_DIM), jnp.float32),
                jax.ShapeDtypeStruct((batch,), jnp.float32)]
    scratch = ([pltpu.VMEM((NUM_FIELDS, rows_per_worker), jnp.int32)]
               + [pltpu.VMEM((rows_per_worker, EMB_DIM), jnp.float32)
                  for _ in range(NUM_FIELDS)]
               + [pltpu.VMEM((rows_per_worker,), jnp.float32)
                  for _ in range(NUM_FIELDS)]
               + [pltpu.VMEM((rows_per_worker,), jnp.float32),
                  pltpu.SemaphoreType.DMA, pltpu.SemaphoreType.DMA])
    fn = pl.kernel(
        functools.partial(_sc_body, rows_per_worker),
        out_type=out_type,
        mesh=plsc.VectorSubcoreMesh(core_axis_name="c", subcore_axis_name="s"),
        scratch_types=scratch,
        compiler_params=pltpu.CompilerParams(use_tc_tiling_on_sc=False),
    )
    return fn(*idxs, *embs, *lins)


def _dotg(a, b, dims):
    return jax.lax.dot_general(a, b, (dims, ((), ())),
                               preferred_element_type=jnp.float32)


def _tc_body(em_ref, e8_ref, xn_ref, ls_ref, s0_ref, w0_ref, b0_ref,
             w1_ref, b1_ref, lnw_ref, lnb_ref, ow_ref, ob_ref, out_ref):
    em = em_ref[...]
    e8 = e8_ref[...]
    xn = xn_ref[...]

    sum_e = _dotg(em, s0_ref[...], ((1,), (0,))) + e8
    sq_sum = (jnp.sum(em * em, axis=1, keepdims=True)
              + jnp.sum(e8 * e8, axis=1, keepdims=True))
    fm = 0.5 * (jnp.sum(sum_e * sum_e, axis=1, keepdims=True) - sq_sum)

    lin = (ls_ref[...] + jnp.sum(xn * lnw_ref[...], axis=1, keepdims=True)
           + lnb_ref[...])

    w0 = w0_ref[...]
    h = (_dotg(em, w0[:, 0:128], ((1,), (1,)))
         + _dotg(e8, w0[:, 128:144], ((1,), (1,)))
         + _dotg(xn, w0[:, 144:147], ((1,), (1,)))
         + b0_ref[...])
    h = jnp.maximum(h, 0.0)
    h = jnp.maximum(_dotg(h, w1_ref[...], ((1,), (1,))) + b1_ref[...], 0.0)

    ow = ow_ref[...]
    logit = (fm * ow[0:1, 0:1] + lin * ow[0:1, 1:2]
             + jnp.sum(h * ow[0:1, 2:34], axis=1, keepdims=True)
             + ob_ref[...])
    out_ref[...] = jax.nn.sigmoid(logit)


def _tc_dense(emain, e8, xn, ls2d, s0, w0, b0, w1, b1, lnw, lnb, ow, ob,
              blk=2048):
    batch = emain.shape[0]

    def rowblock(w):
        return pl.BlockSpec((blk, w), lambda i: (i, 0))

    def whole(shape):
        return pl.BlockSpec(shape, lambda i: tuple(0 for _ in shape))

    in_specs = [rowblock(128), rowblock(EMB_DIM), rowblock(3), rowblock(1),
                whole(s0.shape), whole(w0.shape), whole(b0.shape),
                whole(w1.shape), whole(b1.shape), whole(lnw.shape),
                whole(lnb.shape), whole(ow.shape), whole(ob.shape)]
    return pl.pallas_call(
        _tc_body,
        grid=(batch // blk,),
        in_specs=in_specs,
        out_specs=rowblock(1),
        out_shape=jax.ShapeDtypeStruct((batch, 1), jnp.float32),
    )(emain, e8, xn, ls2d, s0, w0, b0, w1, b1, lnw, lnb, ow, ob)


def kernel(idx_user_id, idx_region, idx_device, idx_gender, idx_banner_id,
           idx_brand, idx_vertical, idx_language, idx_price_tier,
           x_num,
           emb_user_id, emb_region, emb_device, emb_gender, emb_banner_id,
           emb_brand, emb_vertical, emb_language, emb_price_tier,
           lin_user_id, lin_region, lin_device, lin_gender, lin_banner_id,
           lin_brand, lin_vertical, lin_language, lin_price_tier,
           lin_num_W, lin_num_b,
           dnn_W0, dnn_b0, dnn_W1, dnn_b1,
           out_W, out_b):
    idxs = [idx_user_id, idx_region, idx_device, idx_gender, idx_banner_id,
            idx_brand, idx_vertical, idx_language, idx_price_tier]
    idxs = [i.astype(jnp.int32) for i in idxs]
    embs = [emb_user_id, emb_region, emb_device, emb_gender, emb_banner_id,
            emb_brand, emb_vertical, emb_language, emb_price_tier]
    lins = [lin_user_id, lin_region, lin_device, lin_gender, lin_banner_id,
            lin_brand, lin_vertical, lin_language, lin_price_tier]
    lins = [jnp.reshape(l, (-1,)) for l in lins]
    batch = idxs[0].shape[0]

    emain, e8, lsum = _sc_gather(idxs, embs, lins)

    s0 = jnp.tile(jnp.eye(EMB_DIM, dtype=jnp.float32), (8, 1))
    return _tc_dense(
        emain, e8, x_num, jnp.reshape(lsum, (batch, 1)), s0,
        dnn_W0, jnp.reshape(dnn_b0, (1, -1)),
        dnn_W1, jnp.reshape(dnn_b1, (1, -1)),
        lin_num_W, jnp.reshape(lin_num_b, (1, 1)),
        out_W, jnp.reshape(out_b, (1, 1)))
